# baseline (speedup 1.0000x reference)
import functools

import jax
import jax.numpy as jnp
from jax import lax
from jax.experimental import pallas as pl
from jax.experimental.pallas import tpu as pltpu
from jax.experimental.pallas import tpu_sc as plsc

F32 = jnp.float32
I32 = jnp.int32

D = 128
BN = 512
NC = 2
NS = 16
NW = NC * NS
BLK = 128
CAP_BLKS = 96
CAP = CAP_BLKS * BLK
SCAN = 3200
NOFF = 112


def _mm_body(two_inputs, *refs):
    if two_inputs:
        (x1, w1, x2, w2, a_src, a_dst,
         h_ref, as_ref, ad_ref, m_ref, ms, md) = refs
    else:
        (x1, w1, a_src, a_dst,
         h_ref, as_ref, ad_ref, m_ref, ms, md) = refs
    i = pl.program_id(0)
    h = jnp.dot(x1[...], w1[...], preferred_element_type=F32)
    if two_inputs:
        h = h + jnp.dot(x2[...], w2[...], preferred_element_type=F32)
    h_ref[...] = h
    asv = jnp.sum(h * a_src[...], axis=1, keepdims=True)
    adv = jnp.sum(h * a_dst[...], axis=1, keepdims=True)
    as_ref[...] = asv
    ad_ref[...] = adv
    bs = jnp.max(asv)
    bd = jnp.max(adv)

    @pl.when(i == 0)
    def _():
        ms[0] = bs
        md[0] = bd

    @pl.when(i > 0)
    def _():
        ms[0] = jnp.maximum(ms[0], bs)
        md[0] = jnp.maximum(md[0], bd)

    m_ref[...] = jnp.full((1, D), jnp.maximum(ms[0] + md[0], 0.0), F32)


def _mm(x1, w1, a_src, a_dst, x2=None, w2=None):
    npad = x1.shape[0]
    grid = npad // BN
    two = x2 is not None
    ins = [x1, w1] + ([x2, w2] if two else []) + [a_src.reshape(1, D),
                                                 a_dst.reshape(1, D)]
    in_specs = [
        pl.BlockSpec((BN, x1.shape[1]), lambda i: (i, 0)),
        pl.BlockSpec((x1.shape[1], D), lambda i: (0, 0)),
    ]
    if two:
        in_specs += [
            pl.BlockSpec((BN, x2.shape[1]), lambda i: (i, 0)),
            pl.BlockSpec((x2.shape[1], D), lambda i: (0, 0)),
        ]
    in_specs += [pl.BlockSpec((1, D), lambda i: (0, 0)),
                 pl.BlockSpec((1, D), lambda i: (0, 0))]
    out_shape = [
        jax.ShapeDtypeStruct((npad, D), F32),
        jax.ShapeDtypeStruct((npad, 1), F32),
        jax.ShapeDtypeStruct((npad, 1), F32),
        jax.ShapeDtypeStruct((1, D), F32),
    ]
    out_specs = [
        pl.BlockSpec((BN, D), lambda i: (i, 0)),
        pl.BlockSpec((BN, 1), lambda i: (i, 0)),
        pl.BlockSpec((BN, 1), lambda i: (i, 0)),
        pl.BlockSpec((1, D), lambda i: (0, 0)),
    ]
    return pl.pallas_call(
        functools.partial(_mm_body, two),
        grid=(grid,),
        in_specs=in_specs,
        out_specs=out_specs,
        out_shape=out_shape,
        scratch_shapes=[pltpu.SMEM((1,), F32), pltpu.SMEM((1,), F32)],
    )(*ins)


def _norm_body(use_gelu, acc, den, h, as_ref, ad_ref, m_ref, b_ref, o_ref):
    z = as_ref[...] + ad_ref[...]
    z = jnp.where(z > 0.0, z, 0.2 * z)
    exl = jnp.exp(z - m_ref[0:1, 0:1])
    num = acc[...] + exl * h[...]
    dn = den[...] + exl + 1e-16
    out = num / dn + b_ref[...]
    if use_gelu:
        out = 0.5 * out * (1.0 + lax.erf(out * 0.7071067811865476))
    o_ref[...] = out


def _norm(acc, den, h, a_s, a_d, m, bias, use_gelu):
    npad = h.shape[0]
    grid = npad // BN
    return pl.pallas_call(
        functools.partial(_norm_body, use_gelu),
        grid=(grid,),
        in_specs=[
            pl.BlockSpec((BN, D), lambda i: (i, 0)),
            pl.BlockSpec((BN, 1), lambda i: (i, 0)),
            pl.BlockSpec((BN, D), lambda i: (i, 0)),
            pl.BlockSpec((BN, 1), lambda i: (i, 0)),
            pl.BlockSpec((BN, 1), lambda i: (i, 0)),
            pl.BlockSpec((1, D), lambda i: (0, 0)),
            pl.BlockSpec((1, D), lambda i: (0, 0)),
        ],
        out_specs=pl.BlockSpec((BN, D), lambda i: (i, 0)),
        out_shape=jax.ShapeDtypeStruct((npad, D), F32),
    )(acc, den.reshape(npad, 1), h, a_s, a_d, m, bias.reshape(1, D))


def _make_sc_prep(e_total, npad, rows_per_tile):
    nscan = e_total // SCAN
    mesh = plsc.VectorSubcoreMesh(core_axis_name="c", subcore_axis_name="s",
                                  num_cores=NC, num_subcores=NS)

    @functools.partial(
        pl.kernel,
        out_type=[jax.ShapeDtypeStruct((NW, CAP), I32),
                  jax.ShapeDtypeStruct((NW, CAP), I32),
                  jax.ShapeDtypeStruct((NW, NOFF), I32)],
        mesh=mesh,
        compiler_params=pltpu.CompilerParams(needs_layout_passes=False),
        scratch_types=[
            pltpu.VMEM((SCAN,), I32),
            pltpu.VMEM((SCAN,), I32),
            pltpu.VMEM((SCAN,), I32),
            pltpu.VMEM((SCAN,), I32),
            pltpu.VMEM((CAP + 16,), I32),
            pltpu.VMEM((CAP + 16,), I32),
            pltpu.VMEM((CAP + 16,), I32),
            pltpu.VMEM((CAP + 16,), I32),
            pltpu.VMEM((NOFF + 16,), I32),
            pltpu.VMEM((NOFF,), I32),
            pltpu.SMEM((NOFF,), I32),
            pltpu.SemaphoreType.DMA,
            pltpu.SemaphoreType.DMA,
        ],
    )
    def sc_prep(src_ref, dst_ref, ls_ref, ld_ref, off_ref,
                sbufA, dbufA, sbufB, dbufB, lsv, ldv, ls2, ld2,
                cntb, offv, cur, semA, semB):
        cid = lax.axis_index("c")
        sid = lax.axis_index("s")
        wid = sid * NC + cid
        lo = wid * rows_per_tile
        lo16 = jnp.full((16,), lo, I32)
        hi16 = jnp.full((16,), lo + rows_per_tile, I32)
        zero16 = jnp.zeros((16,), I32)
        one0i = (lax.iota(I32, 16) == 0).astype(I32)
        one0b = lax.iota(I32, 16) == 0

        @pl.loop(0, (CAP + 16) // 16)
        def _(i):
            lsv[pl.ds(i * 16, 16)] = zero16
            ldv[pl.ds(i * 16, 16)] = lo16
            ls2[pl.ds(i * 16, 16)] = zero16
            ld2[pl.ds(i * 16, 16)] = lo16

        @pl.loop(0, (NOFF + 16) // 16)
        def _(i):
            cntb[pl.ds(i * 16, 16)] = zero16

        def scan_buf(sb, db, off):
            def step(i, off):
                svs, dvs, msks, pcs = [], [], [], []
                for u in range(4):
                    s16 = sb[pl.ds((i * 4 + u) * 16, 16)]
                    d16 = db[pl.ds((i * 4 + u) * 16, 16)]
                    msk = (d16 >= lo16) & (d16 < hi16)
                    svs.append(s16)
                    dvs.append(d16)
                    msks.append(msk)
                    pcs.append(plsc.all_reduce_population_count(msk)[0])
                o = off
                for u in range(4):
                    plsc.store_compressed(lsv.at[pl.ds(o, 16)], svs[u],
                                          mask=msks[u])
                    plsc.store_compressed(ldv.at[pl.ds(o, 16)], dvs[u],
                                          mask=msks[u])
                    o = o + pcs[u]
                return o

            return lax.fori_loop(0, SCAN // 64, step, off)

        def issue(ci, sb, db, sem):
            base = jnp.minimum(ci, nscan - 1) * SCAN
            pltpu.async_copy(src_ref.at[pl.ds(base, SCAN)], sb, sem)
            pltpu.async_copy(dst_ref.at[pl.ds(base, SCAN)], db, sem)

        def drain(sb, db, sem):
            pltpu.make_async_copy(src_ref.at[pl.ds(0, SCAN)], sb, sem).wait()
            pltpu.make_async_copy(dst_ref.at[pl.ds(0, SCAN)], db, sem).wait()

        issue(0, sbufA, dbufA, semA)

        def scan_pair(h, off):
            c0 = h * 2
            issue(c0 + 1, sbufB, dbufB, semB)
            drain(sbufA, dbufA, semA)
            off = scan_buf(sbufA, dbufA, off)
            issue(c0 + 2, sbufA, dbufA, semA)
            drain(sbufB, dbufB, semB)
            off = scan_buf(sbufB, dbufB, off)
            return off

        cnt = lax.fori_loop(0, nscan // 2, scan_pair, jnp.int32(0))
        drain(sbufA, dbufA, semA)

        nfull = cnt >> 4
        rem = cnt & 15

        @pl.loop(0, nfull)
        def _(g):
            b16 = lsv[pl.ds(g * 16, 16)] >> 7
            for j2 in range(16):
                b = b16[j2]
                plsc.addupdate(cntb.at[pl.ds(b, 16)], one0i)

        b16t = lsv[pl.ds(nfull * 16, 16)] >> 7
        for j2 in range(16):
            @pl.when(j2 < rem)
            def _():
                b = b16t[j2]
                plsc.addupdate(cntb.at[pl.ds(b, 16)], one0i)

        running = jnp.int32(0)
        for cg in range(NOFF // 16):
            v = cntb[pl.ds(cg * 16, 16)]
            cs = plsc.cumsum(v)
            excl = cs - v + jnp.full((16,), running, I32)
            offv[pl.ds(cg * 16, 16)] = excl
            for l in range(16):
                cur[cg * 16 + l] = excl[l]
            running = running + cs[15]

        @pl.loop(0, (cnt + 15) >> 4)
        def _(g):
            s16 = lsv[pl.ds(g * 16, 16)]
            d16 = ldv[pl.ds(g * 16, 16)]
            b16 = s16 >> 7
            nin = jnp.minimum(cnt - g * 16, 16)

            @pl.loop(0, nin)
            def _(j2):
                jb = jnp.full((16,), j2, I32)
                b = jnp.take(b16, jb)[0]
                slot = plsc.fetch_and_add(cur.at[b], 1, subcore_id=sid)
                sv = jnp.take(s16, jb)
                dv = jnp.take(d16, jb)
                plsc.store_compressed(ls2.at[pl.ds(slot, 16)], sv,
                                      mask=one0b)
                plsc.store_compressed(ld2.at[pl.ds(slot, 16)], dv,
                                      mask=one0b)

        pltpu.sync_copy(ls2.at[pl.ds(0, CAP)], ls_ref.at[wid])
        pltpu.sync_copy(ld2.at[pl.ds(0, CAP)], ld_ref.at[wid])
        pltpu.sync_copy(offv, off_ref.at[wid])

    return sc_prep


def _make_sc_edge(npad, rows_per_tile):
    nch = npad // BLK
    mesh = plsc.VectorSubcoreMesh(core_axis_name="c", subcore_axis_name="s",
                                  num_cores=NC, num_subcores=NS)

    @functools.partial(
        pl.kernel,
        out_type=[jax.ShapeDtypeStruct((npad, D), F32),
                  jax.ShapeDtypeStruct((npad,), F32)],
        mesh=mesh,
        compiler_params=pltpu.CompilerParams(needs_layout_passes=False),
        scratch_types=[
            pltpu.VMEM((npad,), F32),
            pltpu.VMEM((npad,), F32),
            pltpu.VMEM((CAP,), I32),
            pltpu.VMEM((CAP,), I32),
            pltpu.VMEM((NOFF,), I32),
            pltpu.VMEM((16,), F32),
            pltpu.VMEM((BLK, D), F32),
            pltpu.VMEM((BLK, D), F32),
            pltpu.VMEM((rows_per_tile, D), F32),
            pltpu.VMEM((rows_per_tile + 16,), F32),
            pltpu.SemaphoreType.DMA,
            pltpu.SemaphoreType.DMA,
        ],
    )
    def sc_edge(h_ref, as_ref, ad_ref, m_ref, ls_ref, ld_ref, off_ref,
                acc_out, den_out,
                asv, adv, srcv, dstv, offv, mv, rowA, rowB, accv, denv,
                semA, semB):
        cid = lax.axis_index("c")
        sid = lax.axis_index("s")
        wid = sid * NC + cid
        lo = wid * rows_per_tile

        zero16 = jnp.zeros((16,), F32)

        @pl.loop(0, rows_per_tile)
        def _(r):
            for k in range(D // 16):
                accv[r, pl.ds(k * 16, 16)] = zero16

        @pl.loop(0, (rows_per_tile + 16) // 16)
        def _(i):
            denv[pl.ds(i * 16, 16)] = zero16

        pltpu.sync_copy(as_ref, asv)
        pltpu.sync_copy(ad_ref, adv)
        pltpu.sync_copy(m_ref, mv)
        pltpu.sync_copy(ls_ref.at[wid], srcv)
        pltpu.sync_copy(ld_ref.at[wid], dstv)
        pltpu.sync_copy(off_ref.at[wid], offv)
        m = mv[...]
        lo16 = jnp.full((16,), lo, I32)
        one0f = (lax.iota(I32, 16) == 0).astype(F32)
        iota16 = lax.iota(I32, 16)

        def issue(c, row, sem):
            cc = jnp.minimum(c, nch - 1)
            pltpu.async_copy(h_ref.at[pl.ds(cc * BLK, BLK)], row, sem)

        def drain(row, sem):
            pltpu.make_async_copy(h_ref.at[pl.ds(0, BLK)], row, sem).wait()

        def process(c, row):
            ov = offv[pl.ds(c, 16)]
            jlo = ov[0]
            jhi = ov[1]
            jlo16 = jnp.full((16,), jlo, I32)
            jhi16 = jnp.full((16,), jhi, I32)
            cb16 = jnp.full((16,), c * BLK, I32)

            @pl.loop(jlo >> 4, (jhi + 15) >> 4)
            def _(g):
                base = g * 16
                s16 = srcv[pl.ds(base, 16)]
                d16 = dstv[pl.ds(base, 16)]
                av = plsc.load_gather(asv, [s16])
                bv = plsc.load_gather(adv, [d16])
                z = av + bv
                z = jnp.where(z > 0.0, z, 0.2 * z)
                ex = jnp.exp(z - m)
                idx = jnp.full((16,), base, I32) + iota16
                msk = (idx >= jlo16) & (idx < jhi16)
                ex = jnp.where(msk, ex, 0.0)
                sl16 = jnp.clip(s16 - cb16, 0, BLK - 1)
                r16 = d16 - lo16
                for j2 in range(16):
                    ev = jnp.take(ex, jnp.full((16,), j2, I32))
                    sl = sl16[j2]
                    r = r16[j2]
                    plsc.addupdate(denv.at[pl.ds(r, 16)], ev * one0f)
                    for k in range(D // 16):
                        plsc.addupdate(accv.at[r, pl.ds(k * 16, 16)],
                                       row[sl, pl.ds(k * 16, 16)] * ev)

        issue(0, rowA, semA)

        @pl.loop(0, nch // 2)
        def _(hh):
            c0 = hh * 2
            issue(c0 + 1, rowB, semB)
            drain(rowA, semA)
            process(c0, rowA)
            issue(c0 + 2, rowA, semA)
            drain(rowB, semB)
            process(c0 + 1, rowB)

        drain(rowA, semA)

        pltpu.sync_copy(accv, acc_out.at[pl.ds(lo, rows_per_tile)])
        pltpu.sync_copy(denv.at[pl.ds(0, rows_per_tile)],
                        den_out.at[pl.ds(lo, rows_per_tile)])

    return sc_edge


def kernel(x, edge_index, W_std, att_src_std, att_dst_std, bias_std,
           W_skip, att_src_skip, att_dst_skip, bias_skip):
    n, d = x.shape
    e = edge_index.shape[1]
    npad = ((n + BN - 1) // BN) * BN
    rows_per_tile = npad // NW

    xp = jnp.pad(x, ((0, npad - n), (0, 0)))

    sc_prep = _make_sc_prep(e, npad, rows_per_tile)
    sc_edge = _make_sc_edge(npad, rows_per_tile)

    ls, ld, offs = sc_prep(edge_index[0], edge_index[1])

    def layer(xin, w, a_src, a_dst, bias, use_gelu, x2=None, w2=None):
        h, a_s, a_d, m = _mm(xin, w, a_src, a_dst, x2=x2, w2=w2)
        acc, den = sc_edge(h, a_s.reshape(npad), a_d.reshape(npad),
                           m[0, :16], ls, ld, offs)
        return _norm(acc, den, h, a_s, a_d, m, bias, use_gelu)

    xc = xp
    for _ in range(4):
        xc = layer(xc, W_std, att_src_std, att_dst_std, bias_std, True)
    out = layer(xp, W_skip[:d], att_src_skip, att_dst_skip, bias_skip,
                False, x2=xc, w2=W_skip[d:])
    return out[:n]


# speedup vs baseline: 1.0167x; 1.0167x over previous
import functools

import jax
import jax.numpy as jnp
from jax import lax
from jax.experimental import pallas as pl
from jax.experimental.pallas import tpu as pltpu
from jax.experimental.pallas import tpu_sc as plsc

F32 = jnp.float32
I32 = jnp.int32

D = 128
BN = 512
NC = 2
NS = 16
NW = NC * NS
BLK = 128
CAP_BLKS = 96
CAP = CAP_BLKS * BLK
SCAN = 3200
NOFF = 112


def _mm_body(two_inputs, *refs):
    if two_inputs:
        (x1, w1, x2, w2, a_src, a_dst,
         h_ref, as_ref, ad_ref, m_ref, ms, md) = refs
    else:
        (x1, w1, a_src, a_dst,
         h_ref, as_ref, ad_ref, m_ref, ms, md) = refs
    i = pl.program_id(0)
    h = jnp.dot(x1[...], w1[...], preferred_element_type=F32)
    if two_inputs:
        h = h + jnp.dot(x2[...], w2[...], preferred_element_type=F32)
    h_ref[...] = h
    asv = jnp.sum(h * a_src[...], axis=1, keepdims=True)
    adv = jnp.sum(h * a_dst[...], axis=1, keepdims=True)
    as_ref[...] = asv
    ad_ref[...] = adv
    bs = jnp.max(asv)
    bd = jnp.max(adv)

    @pl.when(i == 0)
    def _():
        ms[0] = bs
        md[0] = bd

    @pl.when(i > 0)
    def _():
        ms[0] = jnp.maximum(ms[0], bs)
        md[0] = jnp.maximum(md[0], bd)

    m_ref[...] = jnp.full((1, D), jnp.maximum(ms[0] + md[0], 0.0), F32)


def _mm(x1, w1, a_src, a_dst, x2=None, w2=None):
    npad = x1.shape[0]
    grid = npad // BN
    two = x2 is not None
    ins = [x1, w1] + ([x2, w2] if two else []) + [a_src.reshape(1, D),
                                                 a_dst.reshape(1, D)]
    in_specs = [
        pl.BlockSpec((BN, x1.shape[1]), lambda i: (i, 0)),
        pl.BlockSpec((x1.shape[1], D), lambda i: (0, 0)),
    ]
    if two:
        in_specs += [
            pl.BlockSpec((BN, x2.shape[1]), lambda i: (i, 0)),
            pl.BlockSpec((x2.shape[1], D), lambda i: (0, 0)),
        ]
    in_specs += [pl.BlockSpec((1, D), lambda i: (0, 0)),
                 pl.BlockSpec((1, D), lambda i: (0, 0))]
    out_shape = [
        jax.ShapeDtypeStruct((npad, D), F32),
        jax.ShapeDtypeStruct((npad, 1), F32),
        jax.ShapeDtypeStruct((npad, 1), F32),
        jax.ShapeDtypeStruct((1, D), F32),
    ]
    out_specs = [
        pl.BlockSpec((BN, D), lambda i: (i, 0)),
        pl.BlockSpec((BN, 1), lambda i: (i, 0)),
        pl.BlockSpec((BN, 1), lambda i: (i, 0)),
        pl.BlockSpec((1, D), lambda i: (0, 0)),
    ]
    return pl.pallas_call(
        functools.partial(_mm_body, two),
        grid=(grid,),
        in_specs=in_specs,
        out_specs=out_specs,
        out_shape=out_shape,
        scratch_shapes=[pltpu.SMEM((1,), F32), pltpu.SMEM((1,), F32)],
    )(*ins)


def _norm_body(use_gelu, acc, den, h, as_ref, ad_ref, m_ref, b_ref, o_ref):
    z = as_ref[...] + ad_ref[...]
    z = jnp.where(z > 0.0, z, 0.2 * z)
    exl = jnp.exp(z - m_ref[0:1, 0:1])
    num = acc[...] + exl * h[...]
    dn = den[...] + exl + 1e-16
    out = num / dn + b_ref[...]
    if use_gelu:
        out = 0.5 * out * (1.0 + lax.erf(out * 0.7071067811865476))
    o_ref[...] = out


def _norm(acc, den, h, a_s, a_d, m, bias, use_gelu):
    npad = h.shape[0]
    grid = npad // BN
    return pl.pallas_call(
        functools.partial(_norm_body, use_gelu),
        grid=(grid,),
        in_specs=[
            pl.BlockSpec((BN, D), lambda i: (i, 0)),
            pl.BlockSpec((BN, 1), lambda i: (i, 0)),
            pl.BlockSpec((BN, D), lambda i: (i, 0)),
            pl.BlockSpec((BN, 1), lambda i: (i, 0)),
            pl.BlockSpec((BN, 1), lambda i: (i, 0)),
            pl.BlockSpec((1, D), lambda i: (0, 0)),
            pl.BlockSpec((1, D), lambda i: (0, 0)),
        ],
        out_specs=pl.BlockSpec((BN, D), lambda i: (i, 0)),
        out_shape=jax.ShapeDtypeStruct((npad, D), F32),
    )(acc, den.reshape(npad, 1), h, a_s, a_d, m, bias.reshape(1, D))


def _make_sc_prep(e_total, npad, rows_per_tile):
    nscan = e_total // SCAN
    mesh = plsc.VectorSubcoreMesh(core_axis_name="c", subcore_axis_name="s",
                                  num_cores=NC, num_subcores=NS)

    @functools.partial(
        pl.kernel,
        out_type=[jax.ShapeDtypeStruct((NW, CAP), I32),
                  jax.ShapeDtypeStruct((NW, CAP), I32),
                  jax.ShapeDtypeStruct((NW, NOFF), I32)],
        mesh=mesh,
        compiler_params=pltpu.CompilerParams(needs_layout_passes=False),
        scratch_types=[
            pltpu.VMEM((SCAN,), I32),
            pltpu.VMEM((SCAN,), I32),
            pltpu.VMEM((SCAN,), I32),
            pltpu.VMEM((SCAN,), I32),
            pltpu.VMEM((CAP + 16,), I32),
            pltpu.VMEM((CAP + 16,), I32),
            pltpu.VMEM((CAP + 16,), I32),
            pltpu.VMEM((CAP + 16,), I32),
            pltpu.VMEM((NOFF + 16,), I32),
            pltpu.VMEM((NOFF,), I32),
            pltpu.SMEM((NOFF,), I32),
            pltpu.SemaphoreType.DMA,
            pltpu.SemaphoreType.DMA,
        ],
    )
    def sc_prep(src_ref, dst_ref, ls_ref, ld_ref, off_ref,
                sbufA, dbufA, sbufB, dbufB, lsv, ldv, ls2, ld2,
                cntb, offv, cur, semA, semB):
        cid = lax.axis_index("c")
        sid = lax.axis_index("s")
        wid = sid * NC + cid
        lo = wid * rows_per_tile
        lo16 = jnp.full((16,), lo, I32)
        hi16 = jnp.full((16,), lo + rows_per_tile, I32)
        zero16 = jnp.zeros((16,), I32)
        one0i = (lax.iota(I32, 16) == 0).astype(I32)
        one0b = lax.iota(I32, 16) == 0

        @pl.loop(0, (CAP + 16) // 16)
        def _(i):
            lsv[pl.ds(i * 16, 16)] = zero16
            ldv[pl.ds(i * 16, 16)] = lo16
            ls2[pl.ds(i * 16, 16)] = zero16
            ld2[pl.ds(i * 16, 16)] = lo16

        @pl.loop(0, (NOFF + 16) // 16)
        def _(i):
            cntb[pl.ds(i * 16, 16)] = zero16

        def scan_buf(sb, db, off):
            def step(i, off):
                svs, dvs, msks, pcs = [], [], [], []
                for u in range(4):
                    s16 = sb[pl.ds((i * 4 + u) * 16, 16)]
                    d16 = db[pl.ds((i * 4 + u) * 16, 16)]
                    msk = (d16 >= lo16) & (d16 < hi16)
                    svs.append(s16)
                    dvs.append(d16)
                    msks.append(msk)
                    pcs.append(plsc.all_reduce_population_count(msk)[0])
                o = off
                for u in range(4):
                    plsc.store_compressed(lsv.at[pl.ds(o, 16)], svs[u],
                                          mask=msks[u])
                    plsc.store_compressed(ldv.at[pl.ds(o, 16)], dvs[u],
                                          mask=msks[u])
                    o = o + pcs[u]
                return o

            return lax.fori_loop(0, SCAN // 64, step, off)

        def issue(ci, sb, db, sem):
            base = jnp.minimum(ci, nscan - 1) * SCAN
            pltpu.async_copy(src_ref.at[pl.ds(base, SCAN)], sb, sem)
            pltpu.async_copy(dst_ref.at[pl.ds(base, SCAN)], db, sem)

        def drain(sb, db, sem):
            pltpu.make_async_copy(src_ref.at[pl.ds(0, SCAN)], sb, sem).wait()
            pltpu.make_async_copy(dst_ref.at[pl.ds(0, SCAN)], db, sem).wait()

        issue(0, sbufA, dbufA, semA)

        def scan_pair(h, off):
            c0 = h * 2
            issue(c0 + 1, sbufB, dbufB, semB)
            drain(sbufA, dbufA, semA)
            off = scan_buf(sbufA, dbufA, off)
            issue(c0 + 2, sbufA, dbufA, semA)
            drain(sbufB, dbufB, semB)
            off = scan_buf(sbufB, dbufB, off)
            return off

        cnt = lax.fori_loop(0, nscan // 2, scan_pair, jnp.int32(0))
        drain(sbufA, dbufA, semA)

        nfull = cnt >> 4
        rem = cnt & 15

        ones16 = jnp.full((16,), 1, I32)
        iot = lax.iota(I32, 16)

        @pl.loop(0, nfull)
        def _(g):
            b16 = lsv[pl.ds(g * 16, 16)] >> 7
            plsc.addupdate_scatter(cntb, [b16], ones16)

        b16t = lsv[pl.ds(nfull * 16, 16)] >> 7
        rem16 = jnp.full((16,), rem, I32)
        plsc.addupdate_scatter(cntb, [b16t], ones16, mask=iot < rem16)

        running = jnp.int32(0)
        for cg in range(NOFF // 16):
            v = cntb[pl.ds(cg * 16, 16)]
            cs = plsc.cumsum(v)
            excl = cs - v + jnp.full((16,), running, I32)
            offv[pl.ds(cg * 16, 16)] = excl
            for l in range(16):
                cur[cg * 16 + l] = excl[l]
            running = running + cs[15]

        @pl.loop(0, (cnt + 15) >> 4)
        def _(g):
            s16 = lsv[pl.ds(g * 16, 16)]
            d16 = ldv[pl.ds(g * 16, 16)]
            b16 = s16 >> 7
            nin = jnp.minimum(cnt - g * 16, 16)

            @pl.loop(0, nin)
            def _(j2):
                jb = jnp.full((16,), j2, I32)
                b = jnp.take(b16, jb)[0]
                slot = plsc.fetch_and_add(cur.at[b], 1, subcore_id=sid)
                sv = jnp.take(s16, jb)
                dv = jnp.take(d16, jb)
                plsc.store_compressed(ls2.at[pl.ds(slot, 16)], sv,
                                      mask=one0b)
                plsc.store_compressed(ld2.at[pl.ds(slot, 16)], dv,
                                      mask=one0b)

        pltpu.sync_copy(ls2.at[pl.ds(0, CAP)], ls_ref.at[wid])
        pltpu.sync_copy(ld2.at[pl.ds(0, CAP)], ld_ref.at[wid])
        pltpu.sync_copy(offv, off_ref.at[wid])

    return sc_prep


def _make_sc_edge(npad, rows_per_tile):
    nch = npad // BLK
    mesh = plsc.VectorSubcoreMesh(core_axis_name="c", subcore_axis_name="s",
                                  num_cores=NC, num_subcores=NS)

    @functools.partial(
        pl.kernel,
        out_type=[jax.ShapeDtypeStruct((npad, D), F32),
                  jax.ShapeDtypeStruct((npad,), F32)],
        mesh=mesh,
        compiler_params=pltpu.CompilerParams(needs_layout_passes=False),
        scratch_types=[
            pltpu.VMEM((npad,), F32),
            pltpu.VMEM((npad,), F32),
            pltpu.VMEM((CAP,), I32),
            pltpu.VMEM((CAP,), I32),
            pltpu.VMEM((NOFF,), I32),
            pltpu.VMEM((16,), F32),
            pltpu.VMEM((BLK, D), F32),
            pltpu.VMEM((BLK, D), F32),
            pltpu.VMEM((rows_per_tile, D), F32),
            pltpu.VMEM((rows_per_tile + 16,), F32),
            pltpu.SemaphoreType.DMA,
            pltpu.SemaphoreType.DMA,
        ],
    )
    def sc_edge(h_ref, as_ref, ad_ref, m_ref, ls_ref, ld_ref, off_ref,
                acc_out, den_out,
                asv, adv, srcv, dstv, offv, mv, rowA, rowB, accv, denv,
                semA, semB):
        cid = lax.axis_index("c")
        sid = lax.axis_index("s")
        wid = sid * NC + cid
        lo = wid * rows_per_tile

        zero16 = jnp.zeros((16,), F32)

        @pl.loop(0, rows_per_tile)
        def _(r):
            for k in range(D // 16):
                accv[r, pl.ds(k * 16, 16)] = zero16

        @pl.loop(0, (rows_per_tile + 16) // 16)
        def _(i):
            denv[pl.ds(i * 16, 16)] = zero16

        pltpu.sync_copy(as_ref, asv)
        pltpu.sync_copy(ad_ref, adv)
        pltpu.sync_copy(m_ref, mv)
        pltpu.sync_copy(ls_ref.at[wid], srcv)
        pltpu.sync_copy(ld_ref.at[wid], dstv)
        pltpu.sync_copy(off_ref.at[wid], offv)
        m = mv[...]
        lo16 = jnp.full((16,), lo, I32)
        one0f = (lax.iota(I32, 16) == 0).astype(F32)
        iota16 = lax.iota(I32, 16)

        def issue(c, row, sem):
            cc = jnp.minimum(c, nch - 1)
            pltpu.async_copy(h_ref.at[pl.ds(cc * BLK, BLK)], row, sem)

        def drain(row, sem):
            pltpu.make_async_copy(h_ref.at[pl.ds(0, BLK)], row, sem).wait()

        def process(c, row):
            ov = offv[pl.ds(c, 16)]
            jlo = ov[0]
            jhi = ov[1]
            jlo16 = jnp.full((16,), jlo, I32)
            jhi16 = jnp.full((16,), jhi, I32)
            cb16 = jnp.full((16,), c * BLK, I32)

            @pl.loop(jlo >> 4, (jhi + 15) >> 4)
            def _(g):
                base = g * 16
                s16 = srcv[pl.ds(base, 16)]
                d16 = dstv[pl.ds(base, 16)]
                av = plsc.load_gather(asv, [s16])
                bv = plsc.load_gather(adv, [d16])
                z = av + bv
                z = jnp.where(z > 0.0, z, 0.2 * z)
                ex = jnp.exp(z - m)
                idx = jnp.full((16,), base, I32) + iota16
                msk = (idx >= jlo16) & (idx < jhi16)
                ex = jnp.where(msk, ex, 0.0)
                sl16 = jnp.clip(s16 - cb16, 0, BLK - 1)
                r16 = d16 - lo16
                plsc.addupdate_scatter(denv, [r16], ex)
                for j2 in range(16):
                    ev = jnp.take(ex, jnp.full((16,), j2, I32))
                    sl = sl16[j2]
                    r = r16[j2]
                    for k in range(D // 16):
                        plsc.addupdate(accv.at[r, pl.ds(k * 16, 16)],
                                       row[sl, pl.ds(k * 16, 16)] * ev)

        issue(0, rowA, semA)

        @pl.loop(0, nch // 2)
        def _(hh):
            c0 = hh * 2
            issue(c0 + 1, rowB, semB)
            drain(rowA, semA)
            process(c0, rowA)
            issue(c0 + 2, rowA, semA)
            drain(rowB, semB)
            process(c0 + 1, rowB)

        drain(rowA, semA)

        pltpu.sync_copy(accv, acc_out.at[pl.ds(lo, rows_per_tile)])
        pltpu.sync_copy(denv.at[pl.ds(0, rows_per_tile)],
                        den_out.at[pl.ds(lo, rows_per_tile)])

    return sc_edge


def kernel(x, edge_index, W_std, att_src_std, att_dst_std, bias_std,
           W_skip, att_src_skip, att_dst_skip, bias_skip):
    n, d = x.shape
    e = edge_index.shape[1]
    npad = ((n + BN - 1) // BN) * BN
    rows_per_tile = npad // NW

    xp = jnp.pad(x, ((0, npad - n), (0, 0)))

    sc_prep = _make_sc_prep(e, npad, rows_per_tile)
    sc_edge = _make_sc_edge(npad, rows_per_tile)

    ls, ld, offs = sc_prep(edge_index[0], edge_index[1])

    def layer(xin, w, a_src, a_dst, bias, use_gelu, x2=None, w2=None):
        h, a_s, a_d, m = _mm(xin, w, a_src, a_dst, x2=x2, w2=w2)
        acc, den = sc_edge(h, a_s.reshape(npad), a_d.reshape(npad),
                           m[0, :16], ls, ld, offs)
        return _norm(acc, den, h, a_s, a_d, m, bias, use_gelu)

    xc = xp
    for _ in range(4):
        xc = layer(xc, W_std, att_src_std, att_dst_std, bias_std, True)
    out = layer(xp, W_skip[:d], att_src_skip, att_dst_skip, bias_skip,
                False, x2=xc, w2=W_skip[d:])
    return out[:n]


# speedup vs baseline: 1.1006x; 1.0826x over previous
import functools

import jax
import jax.numpy as jnp
from jax import lax
from jax.experimental import pallas as pl
from jax.experimental.pallas import tpu as pltpu
from jax.experimental.pallas import tpu_sc as plsc

F32 = jnp.float32
I32 = jnp.int32

D = 128
BN = 512
NC = 2
NS = 16
NW = NC * NS
BLK = 128
CAP_BLKS = 96
CAP = CAP_BLKS * BLK
SCAN = 3200
NOFF = 112


def _mm_body(two_inputs, *refs):
    if two_inputs:
        (x1, w1, x2, w2, a_src, a_dst,
         h_ref, as_ref, ad_ref, m_ref, ms, md) = refs
    else:
        (x1, w1, a_src, a_dst,
         h_ref, as_ref, ad_ref, m_ref, ms, md) = refs
    i = pl.program_id(0)
    h = jnp.dot(x1[...], w1[...], preferred_element_type=F32)
    if two_inputs:
        h = h + jnp.dot(x2[...], w2[...], preferred_element_type=F32)
    h_ref[...] = h
    asv = jnp.sum(h * a_src[...], axis=1, keepdims=True)
    adv = jnp.sum(h * a_dst[...], axis=1, keepdims=True)
    as_ref[...] = asv
    ad_ref[...] = adv
    bs = jnp.max(asv)
    bd = jnp.max(adv)

    @pl.when(i == 0)
    def _():
        ms[0] = bs
        md[0] = bd

    @pl.when(i > 0)
    def _():
        ms[0] = jnp.maximum(ms[0], bs)
        md[0] = jnp.maximum(md[0], bd)

    m_ref[...] = jnp.full((1, D), jnp.maximum(ms[0] + md[0], 0.0), F32)


def _mm(x1, w1, a_src, a_dst, x2=None, w2=None):
    npad = x1.shape[0]
    grid = npad // BN
    two = x2 is not None
    ins = [x1, w1] + ([x2, w2] if two else []) + [a_src.reshape(1, D),
                                                 a_dst.reshape(1, D)]
    in_specs = [
        pl.BlockSpec((BN, x1.shape[1]), lambda i: (i, 0)),
        pl.BlockSpec((x1.shape[1], D), lambda i: (0, 0)),
    ]
    if two:
        in_specs += [
            pl.BlockSpec((BN, x2.shape[1]), lambda i: (i, 0)),
            pl.BlockSpec((x2.shape[1], D), lambda i: (0, 0)),
        ]
    in_specs += [pl.BlockSpec((1, D), lambda i: (0, 0)),
                 pl.BlockSpec((1, D), lambda i: (0, 0))]
    out_shape = [
        jax.ShapeDtypeStruct((npad, D), F32),
        jax.ShapeDtypeStruct((npad, 1), F32),
        jax.ShapeDtypeStruct((npad, 1), F32),
        jax.ShapeDtypeStruct((1, D), F32),
    ]
    out_specs = [
        pl.BlockSpec((BN, D), lambda i: (i, 0)),
        pl.BlockSpec((BN, 1), lambda i: (i, 0)),
        pl.BlockSpec((BN, 1), lambda i: (i, 0)),
        pl.BlockSpec((1, D), lambda i: (0, 0)),
    ]
    return pl.pallas_call(
        functools.partial(_mm_body, two),
        grid=(grid,),
        in_specs=in_specs,
        out_specs=out_specs,
        out_shape=out_shape,
        scratch_shapes=[pltpu.SMEM((1,), F32), pltpu.SMEM((1,), F32)],
    )(*ins)


def _norm_body(use_gelu, acc, den, h, as_ref, ad_ref, m_ref, b_ref, o_ref):
    z = as_ref[...] + ad_ref[...]
    z = jnp.where(z > 0.0, z, 0.2 * z)
    exl = jnp.exp(z - m_ref[0:1, 0:1])
    num = acc[...] + exl * h[...]
    dn = den[...] + exl + 1e-16
    out = num / dn + b_ref[...]
    if use_gelu:
        out = 0.5 * out * (1.0 + lax.erf(out * 0.7071067811865476))
    o_ref[...] = out


def _norm(acc, den, h, a_s, a_d, m, bias, use_gelu):
    npad = h.shape[0]
    grid = npad // BN
    return pl.pallas_call(
        functools.partial(_norm_body, use_gelu),
        grid=(grid,),
        in_specs=[
            pl.BlockSpec((BN, D), lambda i: (i, 0)),
            pl.BlockSpec((BN, 1), lambda i: (i, 0)),
            pl.BlockSpec((BN, D), lambda i: (i, 0)),
            pl.BlockSpec((BN, 1), lambda i: (i, 0)),
            pl.BlockSpec((BN, 1), lambda i: (i, 0)),
            pl.BlockSpec((1, D), lambda i: (0, 0)),
            pl.BlockSpec((1, D), lambda i: (0, 0)),
        ],
        out_specs=pl.BlockSpec((BN, D), lambda i: (i, 0)),
        out_shape=jax.ShapeDtypeStruct((npad, D), F32),
    )(acc, den.reshape(npad, 1), h, a_s, a_d, m, bias.reshape(1, D))


def _make_sc_prep(e_total, npad, rows_per_tile):
    nscan = e_total // SCAN
    mesh = plsc.VectorSubcoreMesh(core_axis_name="c", subcore_axis_name="s",
                                  num_cores=NC, num_subcores=NS)

    @functools.partial(
        pl.kernel,
        out_type=[jax.ShapeDtypeStruct((NW, CAP), I32),
                  jax.ShapeDtypeStruct((NW, CAP), I32),
                  jax.ShapeDtypeStruct((NW, NOFF), I32)],
        mesh=mesh,
        compiler_params=pltpu.CompilerParams(needs_layout_passes=False),
        scratch_types=[
            pltpu.VMEM((SCAN,), I32),
            pltpu.VMEM((SCAN,), I32),
            pltpu.VMEM((SCAN,), I32),
            pltpu.VMEM((SCAN,), I32),
            pltpu.VMEM((CAP + 16,), I32),
            pltpu.VMEM((CAP + 16,), I32),
            pltpu.VMEM((CAP + 16,), I32),
            pltpu.VMEM((CAP + 16,), I32),
            pltpu.VMEM((NOFF + 16,), I32),
            pltpu.VMEM((NOFF,), I32),
            pltpu.VMEM((NOFF,), I32),
            pltpu.SemaphoreType.DMA,
            pltpu.SemaphoreType.DMA,
        ],
    )
    def sc_prep(src_ref, dst_ref, ls_ref, ld_ref, off_ref,
                sbufA, dbufA, sbufB, dbufB, lsv, ldv, ls2, ld2,
                cntb, offv, cur, semA, semB):
        cid = lax.axis_index("c")
        sid = lax.axis_index("s")
        wid = sid * NC + cid
        lo = wid * rows_per_tile
        lo16 = jnp.full((16,), lo, I32)
        hi16 = jnp.full((16,), lo + rows_per_tile, I32)
        zero16 = jnp.zeros((16,), I32)
        one0i = (lax.iota(I32, 16) == 0).astype(I32)
        one0b = lax.iota(I32, 16) == 0

        @pl.loop(0, (CAP + 16) // 16)
        def _(i):
            lsv[pl.ds(i * 16, 16)] = zero16
            ldv[pl.ds(i * 16, 16)] = lo16
            ls2[pl.ds(i * 16, 16)] = zero16
            ld2[pl.ds(i * 16, 16)] = lo16

        @pl.loop(0, (NOFF + 16) // 16)
        def _(i):
            cntb[pl.ds(i * 16, 16)] = zero16

        def scan_buf(sb, db, off):
            def step(i, off):
                svs, dvs, msks, pcs = [], [], [], []
                for u in range(4):
                    s16 = sb[pl.ds((i * 4 + u) * 16, 16)]
                    d16 = db[pl.ds((i * 4 + u) * 16, 16)]
                    msk = (d16 >= lo16) & (d16 < hi16)
                    svs.append(s16)
                    dvs.append(d16)
                    msks.append(msk)
                    pcs.append(plsc.all_reduce_population_count(msk)[0])
                o = off
                for u in range(4):
                    plsc.store_compressed(lsv.at[pl.ds(o, 16)], svs[u],
                                          mask=msks[u])
                    plsc.store_compressed(ldv.at[pl.ds(o, 16)], dvs[u],
                                          mask=msks[u])
                    o = o + pcs[u]
                return o

            return lax.fori_loop(0, SCAN // 64, step, off)

        def issue(ci, sb, db, sem):
            base = jnp.minimum(ci, nscan - 1) * SCAN
            pltpu.async_copy(src_ref.at[pl.ds(base, SCAN)], sb, sem)
            pltpu.async_copy(dst_ref.at[pl.ds(base, SCAN)], db, sem)

        def drain(sb, db, sem):
            pltpu.make_async_copy(src_ref.at[pl.ds(0, SCAN)], sb, sem).wait()
            pltpu.make_async_copy(dst_ref.at[pl.ds(0, SCAN)], db, sem).wait()

        issue(0, sbufA, dbufA, semA)

        def scan_pair(h, off):
            c0 = h * 2
            issue(c0 + 1, sbufB, dbufB, semB)
            drain(sbufA, dbufA, semA)
            off = scan_buf(sbufA, dbufA, off)
            issue(c0 + 2, sbufA, dbufA, semA)
            drain(sbufB, dbufB, semB)
            off = scan_buf(sbufB, dbufB, off)
            return off

        cnt = lax.fori_loop(0, nscan // 2, scan_pair, jnp.int32(0))
        drain(sbufA, dbufA, semA)

        nfull = cnt >> 4
        rem = cnt & 15

        ones16 = jnp.full((16,), 1, I32)
        iot = lax.iota(I32, 16)

        @pl.loop(0, nfull)
        def _(g):
            b16 = lsv[pl.ds(g * 16, 16)] >> 7
            plsc.addupdate_scatter(cntb, [b16], ones16)

        b16t = lsv[pl.ds(nfull * 16, 16)] >> 7
        rem16 = jnp.full((16,), rem, I32)
        plsc.addupdate_scatter(cntb, [b16t], ones16, mask=iot < rem16)

        running = jnp.int32(0)
        for cg in range(NOFF // 16):
            v = cntb[pl.ds(cg * 16, 16)]
            cs = plsc.cumsum(v)
            excl = cs - v + jnp.full((16,), running, I32)
            offv[pl.ds(cg * 16, 16)] = excl
            cur[pl.ds(cg * 16, 16)] = excl
            running = running + cs[15]

        cur[pl.ds(NOFF - 16, 16)] = jnp.full((16,), CAP, I32)

        pad16 = jnp.full((16,), NOFF - 1, I32)
        cnt16 = jnp.full((16,), cnt, I32)
        fifteen = jnp.full((16,), 15, I32)

        @pl.loop(0, (cnt + 15) >> 4)
        def _(g):
            s16 = lsv[pl.ds(g * 16, 16)]
            d16 = ldv[pl.ds(g * 16, 16)]
            valid = (jnp.full((16,), g * 16, I32) + iot) < cnt16
            b16 = jnp.where(valid, s16 >> 7, pad16)
            srt_b, perm = plsc.sort_key_val(b16, iot)
            prev = jnp.take(srt_b, jnp.maximum(iot - 1, 0))
            start = (iot == 0) | (srt_b != prev)
            startpos = plsc.cummax(jnp.where(start, iot, 0))
            rank = iot - startpos
            curs = plsc.load_gather(cur, [srt_b])
            slots = curs + rank
            plsc.store_scatter(ls2, [slots], jnp.take(s16, perm))
            plsc.store_scatter(ld2, [slots], jnp.take(d16, perm))
            nxt = jnp.take(srt_b, jnp.minimum(iot + 1, fifteen))
            end = (iot == 15) | (srt_b != nxt)
            plsc.store_scatter(cur, [srt_b], slots + 1, mask=end)

        pltpu.sync_copy(ls2.at[pl.ds(0, CAP)], ls_ref.at[wid])
        pltpu.sync_copy(ld2.at[pl.ds(0, CAP)], ld_ref.at[wid])
        pltpu.sync_copy(offv, off_ref.at[wid])

    return sc_prep


def _make_sc_edge(npad, rows_per_tile):
    nch = npad // BLK
    mesh = plsc.VectorSubcoreMesh(core_axis_name="c", subcore_axis_name="s",
                                  num_cores=NC, num_subcores=NS)

    @functools.partial(
        pl.kernel,
        out_type=[jax.ShapeDtypeStruct((npad, D), F32),
                  jax.ShapeDtypeStruct((npad,), F32)],
        mesh=mesh,
        compiler_params=pltpu.CompilerParams(needs_layout_passes=False),
        scratch_types=[
            pltpu.VMEM((npad,), F32),
            pltpu.VMEM((npad,), F32),
            pltpu.VMEM((CAP,), I32),
            pltpu.VMEM((CAP,), I32),
            pltpu.VMEM((NOFF,), I32),
            pltpu.VMEM((16,), F32),
            pltpu.VMEM((BLK, D), F32),
            pltpu.VMEM((BLK, D), F32),
            pltpu.VMEM((rows_per_tile, D), F32),
            pltpu.VMEM((rows_per_tile + 16,), F32),
            pltpu.SemaphoreType.DMA,
            pltpu.SemaphoreType.DMA,
        ],
    )
    def sc_edge(h_ref, as_ref, ad_ref, m_ref, ls_ref, ld_ref, off_ref,
                acc_out, den_out,
                asv, adv, srcv, dstv, offv, mv, rowA, rowB, accv, denv,
                semA, semB):
        cid = lax.axis_index("c")
        sid = lax.axis_index("s")
        wid = sid * NC + cid
        lo = wid * rows_per_tile

        zero16 = jnp.zeros((16,), F32)

        @pl.loop(0, rows_per_tile)
        def _(r):
            for k in range(D // 16):
                accv[r, pl.ds(k * 16, 16)] = zero16

        @pl.loop(0, (rows_per_tile + 16) // 16)
        def _(i):
            denv[pl.ds(i * 16, 16)] = zero16

        pltpu.sync_copy(as_ref, asv)
        pltpu.sync_copy(ad_ref, adv)
        pltpu.sync_copy(m_ref, mv)
        pltpu.sync_copy(ls_ref.at[wid], srcv)
        pltpu.sync_copy(ld_ref.at[wid], dstv)
        pltpu.sync_copy(off_ref.at[wid], offv)
        m = mv[...]
        lo16 = jnp.full((16,), lo, I32)
        one0f = (lax.iota(I32, 16) == 0).astype(F32)
        iota16 = lax.iota(I32, 16)

        def issue(c, row, sem):
            cc = jnp.minimum(c, nch - 1)
            pltpu.async_copy(h_ref.at[pl.ds(cc * BLK, BLK)], row, sem)

        def drain(row, sem):
            pltpu.make_async_copy(h_ref.at[pl.ds(0, BLK)], row, sem).wait()

        def process(c, row):
            ov = offv[pl.ds(c, 16)]
            jlo = ov[0]
            jhi = ov[1]
            jlo16 = jnp.full((16,), jlo, I32)
            jhi16 = jnp.full((16,), jhi, I32)
            cb16 = jnp.full((16,), c * BLK, I32)

            @pl.loop(jlo >> 4, (jhi + 15) >> 4)
            def _(g):
                base = g * 16
                s16 = srcv[pl.ds(base, 16)]
                d16 = dstv[pl.ds(base, 16)]
                av = plsc.load_gather(asv, [s16])
                bv = plsc.load_gather(adv, [d16])
                z = av + bv
                z = jnp.where(z > 0.0, z, 0.2 * z)
                ex = jnp.exp(z - m)
                idx = jnp.full((16,), base, I32) + iota16
                msk = (idx >= jlo16) & (idx < jhi16)
                ex = jnp.where(msk, ex, 0.0)
                sl16 = jnp.clip(s16 - cb16, 0, BLK - 1)
                r16 = d16 - lo16
                plsc.addupdate_scatter(denv, [r16], ex)
                for j2 in range(16):
                    ev = jnp.take(ex, jnp.full((16,), j2, I32))
                    sl = sl16[j2]
                    r = r16[j2]
                    for k in range(D // 16):
                        plsc.addupdate(accv.at[r, pl.ds(k * 16, 16)],
                                       row[sl, pl.ds(k * 16, 16)] * ev)

        issue(0, rowA, semA)

        @pl.loop(0, nch // 2)
        def _(hh):
            c0 = hh * 2
            issue(c0 + 1, rowB, semB)
            drain(rowA, semA)
            process(c0, rowA)
            issue(c0 + 2, rowA, semA)
            drain(rowB, semB)
            process(c0 + 1, rowB)

        drain(rowA, semA)

        pltpu.sync_copy(accv, acc_out.at[pl.ds(lo, rows_per_tile)])
        pltpu.sync_copy(denv.at[pl.ds(0, rows_per_tile)],
                        den_out.at[pl.ds(lo, rows_per_tile)])

    return sc_edge


def kernel(x, edge_index, W_std, att_src_std, att_dst_std, bias_std,
           W_skip, att_src_skip, att_dst_skip, bias_skip):
    n, d = x.shape
    e = edge_index.shape[1]
    npad = ((n + BN - 1) // BN) * BN
    rows_per_tile = npad // NW

    xp = jnp.pad(x, ((0, npad - n), (0, 0)))

    sc_prep = _make_sc_prep(e, npad, rows_per_tile)
    sc_edge = _make_sc_edge(npad, rows_per_tile)

    ls, ld, offs = sc_prep(edge_index[0], edge_index[1])

    def layer(xin, w, a_src, a_dst, bias, use_gelu, x2=None, w2=None):
        h, a_s, a_d, m = _mm(xin, w, a_src, a_dst, x2=x2, w2=w2)
        acc, den = sc_edge(h, a_s.reshape(npad), a_d.reshape(npad),
                           m[0, :16], ls, ld, offs)
        return _norm(acc, den, h, a_s, a_d, m, bias, use_gelu)

    xc = xp
    for _ in range(4):
        xc = layer(xc, W_std, att_src_std, att_dst_std, bias_std, True)
    out = layer(xp, W_skip[:d], att_src_skip, att_dst_skip, bias_skip,
                False, x2=xc, w2=W_skip[d:])
    return out[:n]
